# baseline (speedup 1.0000x reference)
import functools

import jax
import jax.numpy as jnp
from jax import lax
from jax.experimental import pallas as pl
from jax.experimental.pallas import tpu as pltpu
from jax.experimental.pallas import tpu_sc as plsc

EMBED_DIM = 128
MAX_TIMESTEPS = 1000
BATCH = 16384

B_SC = 8192
B_TC = BATCH - B_SC

_info = plsc.get_sparse_core_info()
_NC, _NS = _info.num_cores, _info.num_subcores
_NW = _NC * _NS
_B_PER_W = B_SC // _NW

_mesh = plsc.VectorSubcoreMesh(core_axis_name="c", subcore_axis_name="s")

_CHUNK = 128
_NCHUNK = _B_PER_W // _CHUNK


@functools.partial(
    pl.kernel,
    mesh=_mesh,
    out_type=jax.ShapeDtypeStruct((B_SC, EMBED_DIM), jnp.float32),
    scratch_types=[
        pltpu.VMEM_SHARED((MAX_TIMESTEPS, EMBED_DIM), jnp.float32),
        pltpu.VMEM((_NCHUNK, _CHUNK), jnp.int32),
        [pltpu.VMEM((_CHUNK, EMBED_DIM), jnp.float32) for _ in range(_NCHUNK)],
        pltpu.SemaphoreType.DMA,
        pltpu.SemaphoreType.DMA,
    ],
)
def _sc_gather(ts_hbm, pe_hbm, out_hbm, pe_sh, idx_v, bufs, gsem, wsem):
    cid = lax.axis_index("c")
    sid = lax.axis_index("s")
    wid = sid * _NC + cid
    base = wid * _B_PER_W

    @pl.when(sid < 15)
    def _stage():
        pltpu.sync_copy(
            pe_hbm.at[pl.ds(sid * 64, 64)], pe_sh.at[pl.ds(sid * 64, 64)]
        )

    @pl.when(sid == 15)
    def _stage_tail():
        pltpu.sync_copy(pe_hbm.at[pl.ds(960, 40)], pe_sh.at[pl.ds(960, 40)])

    idx_copies = [
        pltpu.async_copy(
            ts_hbm.at[pl.ds(base + i * _CHUNK, _CHUNK)], idx_v.at[i], wsem
        )
        for i in range(_NCHUNK)
    ]
    for c in idx_copies:
        c.wait()
    plsc.subcore_barrier()
    gathers = [
        pltpu.async_copy(pe_sh.at[idx_v.at[i]], bufs[i], gsem)
        for i in range(_NCHUNK)
    ]
    writes = []
    for i in range(_NCHUNK):
        gathers[i].wait()
        writes.append(
            pltpu.async_copy(
                bufs[i], out_hbm.at[pl.ds(base + i * _CHUNK, _CHUNK)], wsem
            )
        )
    for w in writes:
        w.wait()


_VPAD = 1024
_BM = 2048


def _tc_body(ts_ref, pe_ref, out_ref):
    ts = ts_ref[...]
    kiota = jax.lax.broadcasted_iota(jnp.int32, (_BM, _VPAD), 1)
    onehot = (kiota == ts[:, None]).astype(jnp.bfloat16)
    out_ref[...] = jnp.dot(
        onehot, pe_ref[...], preferred_element_type=jnp.float32
    )


def _tc_gather(ts, pe_padded):
    return pl.pallas_call(
        _tc_body,
        grid=(B_TC // _BM,),
        in_specs=[
            pl.BlockSpec((_BM,), lambda i: (i,)),
            pl.BlockSpec((_VPAD, EMBED_DIM), lambda i: (0, 0)),
        ],
        out_specs=pl.BlockSpec((_BM, EMBED_DIM), lambda i: (i, 0)),
        out_shape=jax.ShapeDtypeStruct((B_TC, EMBED_DIM), jnp.float32),
    )(ts, pe_padded)


def kernel(timesteps, pe):
    ts32 = timesteps.astype(jnp.int32)
    pe_padded = jnp.zeros((_VPAD, EMBED_DIM), jnp.bfloat16).at[:MAX_TIMESTEPS].set(
        pe.astype(jnp.bfloat16)
    )
    sc_out = _sc_gather(ts32[:B_SC], pe)
    tc_out = _tc_gather(ts32[B_SC:], pe_padded)
    return jnp.concatenate([sc_out, tc_out], axis=0)


# speedup vs baseline: 1.3284x; 1.3284x over previous
import functools

import jax
import jax.numpy as jnp
from jax import lax
from jax.experimental import pallas as pl
from jax.experimental.pallas import tpu as pltpu
from jax.experimental.pallas import tpu_sc as plsc

EMBED_DIM = 128
MAX_TIMESTEPS = 1000
VPAD = 1024
BATCH = 16384

_info = plsc.get_sparse_core_info()
_NC, _NS = _info.num_cores, _info.num_subcores
_NW = _NC * _NS
_B_PER_W = BATCH // _NW
_ROWS_PER_TILE = VPAD // _NS

_mesh = plsc.VectorSubcoreMesh(core_axis_name="c", subcore_axis_name="s")

_CHUNK = 128
_NCHUNK = _B_PER_W // _CHUNK


@functools.partial(
    pl.kernel,
    mesh=_mesh,
    out_type=jax.ShapeDtypeStruct((BATCH, EMBED_DIM), jnp.float32),
    scratch_types=[
        pltpu.VMEM_SHARED((VPAD, EMBED_DIM), jnp.float32),
        pltpu.VMEM((_NCHUNK, _CHUNK), jnp.int32),
        [pltpu.VMEM((_CHUNK, EMBED_DIM), jnp.float32) for _ in range(_NCHUNK)],
        pltpu.SemaphoreType.DMA,
        pltpu.SemaphoreType.DMA,
        pltpu.SemaphoreType.DMA,
        pltpu.SemaphoreType.DMA,
        pltpu.SemaphoreType.DMA,
    ],
)
def _gather_kernel(
    ts_hbm, pe_hbm, out_hbm, pe_sh, idx_v, bufs, gsem, wsem, ssem, isem, hsem
):
    cid = lax.axis_index("c")
    sid = lax.axis_index("s")
    wid = sid * _NC + cid
    base = wid * _B_PER_W

    idx0 = pltpu.async_copy(ts_hbm.at[pl.ds(base, _CHUNK)], idx_v.at[0], isem)
    idx_rest = [
        pltpu.async_copy(
            ts_hbm.at[pl.ds(base + i * _CHUNK, _CHUNK)], idx_v.at[i], wsem
        )
        for i in range(1, _NCHUNK)
    ]
    stage = pltpu.async_copy(
        pe_hbm.at[pl.ds(sid * _ROWS_PER_TILE, _ROWS_PER_TILE)],
        pe_sh.at[pl.ds(sid * _ROWS_PER_TILE, _ROWS_PER_TILE)],
        ssem,
    )
    idx0.wait()
    gathers = [pltpu.async_copy(pe_hbm.at[idx_v.at[0]], bufs[0], hsem)]
    for c in idx_rest:
        c.wait()
    stage.wait()
    plsc.subcore_barrier()
    gathers += [
        pltpu.async_copy(pe_sh.at[idx_v.at[i]], bufs[i], gsem)
        for i in range(1, _NCHUNK)
    ]
    writes = []
    for i in range(_NCHUNK):
        gathers[i].wait()
        writes.append(
            pltpu.async_copy(
                bufs[i], out_hbm.at[pl.ds(base + i * _CHUNK, _CHUNK)], wsem
            )
        )
    for w in writes:
        w.wait()


def kernel(timesteps, pe):
    pe_padded = jnp.zeros((VPAD, EMBED_DIM), jnp.float32).at[:MAX_TIMESTEPS].set(pe)
    return _gather_kernel(timesteps.astype(jnp.int32), pe_padded)
